# baseline (speedup 1.0000x reference)
import functools

import jax
import jax.numpy as jnp
from jax import lax
from jax.experimental import pallas as pl
from jax.experimental.pallas import tpu as pltpu
from jax.experimental.pallas import tpu_sc as plsc

HIDDEN = 2048
FFN = 2048
EXPERTS = 8
TOPK = 2
ROW_TILE = 128
N_SLOTS = 0

_NUM_WORKERS = 32
_GATHER_CHUNK = 16


def _sc_gather_rows(table, idx):
    n_rows = idx.shape[0]
    row_shape = table.shape[1:]
    npw = n_rows // _NUM_WORKERS
    assert npw * _NUM_WORKERS == n_rows and npw % 8 == 0
    nch = npw // _GATHER_CHUNK
    assert nch * _GATHER_CHUNK == npw and nch % 2 == 0

    mesh = plsc.VectorSubcoreMesh(core_axis_name="c", subcore_axis_name="s")

    def body(table_hbm, idx_hbm, out_hbm, idx_v, rows_a, rows_b, gsem, osem):
        wid = lax.axis_index("s") * 2 + lax.axis_index("c")
        base = wid * npw
        pltpu.sync_copy(idx_hbm.at[pl.ds(base, npw)], idx_v)

        @pl.loop(0, nch, step=2)
        def _pair(c):
            off0 = base + c * _GATHER_CHUNK
            off1 = off0 + _GATHER_CHUNK
            g0 = pltpu.async_copy(
                table_hbm.at[idx_v.at[pl.ds(c * _GATHER_CHUNK, _GATHER_CHUNK)]],
                rows_a, gsem)
            g1 = pltpu.async_copy(
                table_hbm.at[idx_v.at[pl.ds((c + 1) * _GATHER_CHUNK,
                                            _GATHER_CHUNK)]],
                rows_b, gsem)
            g0.wait()
            o0 = pltpu.async_copy(rows_a, out_hbm.at[pl.ds(off0, _GATHER_CHUNK)],
                                  osem)
            g1.wait()
            o1 = pltpu.async_copy(rows_b, out_hbm.at[pl.ds(off1, _GATHER_CHUNK)],
                                  osem)
            o0.wait()
            o1.wait()

    return pl.kernel(
        body,
        out_type=jax.ShapeDtypeStruct((n_rows,) + row_shape, table.dtype),
        mesh=mesh,
        scratch_types=[
            pltpu.VMEM((npw,), jnp.int32),
            pltpu.VMEM((_GATHER_CHUNK,) + row_shape, table.dtype),
            pltpu.VMEM((_GATHER_CHUNK,) + row_shape, table.dtype),
            pltpu.SemaphoreType.DMA,
            pltpu.SemaphoreType.DMA,
        ],
    )(table, idx)


F_SPLIT = 2
F_CHUNK = FFN // F_SPLIT


def _router_body(x_ref, wr_ref, ranks_ref, te_ref, gate_ref, stats_ref,
                 accc, accp, accf, accz):
    i = pl.program_id(0)

    @pl.when(i == 0)
    def _init():
        accc[...] = jnp.zeros_like(accc)
        accp[...] = jnp.zeros_like(accp)
        accf[...] = jnp.zeros_like(accf)
        accz[...] = jnp.zeros_like(accz)

    rows = x_ref.shape[0]
    logits = jnp.dot(x_ref[...], wr_ref[...],
                     preferred_element_type=jnp.float32)
    iota = lax.broadcasted_iota(jnp.int32, (rows, EXPERTS), 1)
    m1 = jnp.max(logits, axis=1, keepdims=True)
    e0 = jnp.min(jnp.where(logits == m1, iota, EXPERTS), axis=1, keepdims=True)
    oh0 = iota == e0
    masked = jnp.where(oh0, -jnp.inf, logits)
    m2 = jnp.max(masked, axis=1, keepdims=True)
    e1 = jnp.min(jnp.where(masked == m2, iota, EXPERTS), axis=1, keepdims=True)
    oh1 = iota == e1
    ed = jnp.exp(m2 - m1)
    g0 = 1.0 / (1.0 + ed)
    g1 = ed / (1.0 + ed)
    ex = jnp.exp(logits - m1)
    sex = jnp.sum(ex, axis=1, keepdims=True)
    lse = m1 + jnp.log(sex)
    r_iota = lax.broadcasted_iota(jnp.int32, (rows, rows), 0)
    c_iota = lax.broadcasted_iota(jnp.int32, (rows, rows), 1)
    tri = (c_iota < r_iota).astype(jnp.float32)
    oh0f = oh0.astype(jnp.float32)
    oh1f = oh1.astype(jnp.float32)
    prior = accc[...] + jnp.dot(tri, oh0f + oh1f,
                                preferred_element_type=jnp.float32)
    rank0 = jnp.sum(prior * oh0f, axis=1, keepdims=True)
    rank1 = jnp.sum(prior * oh1f, axis=1, keepdims=True)
    ranks_ref[0] = jnp.concatenate([rank0, rank1], axis=1).astype(jnp.int32)
    te_ref[0] = jnp.concatenate([e0, e1], axis=1)
    gate_ref[0] = jnp.concatenate([g0, g1], axis=1)
    accc[...] += jnp.sum(oh0f + oh1f, axis=0, keepdims=True)
    accp[...] += jnp.sum(ex / sex, axis=0, keepdims=True)
    accf[...] += jnp.sum(oh0f + jnp.where(g1 > 0, oh1f, 0.0), axis=0,
                         keepdims=True)
    accz[...] += jnp.sum(lse * lse).reshape(1, 1)

    @pl.when(i == pl.num_programs(0) - 1)
    def _fin():
        stats_ref[...] = jnp.concatenate(
            [accc[...], accp[...], accf[...],
             jnp.broadcast_to(accz[...], (1, EXPERTS))], axis=0)


def _router(xf, w_router, t):
    rt = 512
    tiles = t // rt
    return pl.pallas_call(
        _router_body,
        grid=(tiles,),
        in_specs=[
            pl.BlockSpec((rt, HIDDEN), lambda i: (i, 0)),
            pl.BlockSpec((HIDDEN, EXPERTS), lambda i: (0, 0)),
        ],
        out_specs=[
            pl.BlockSpec((1, rt, TOPK), lambda i: (i, 0, 0)),
            pl.BlockSpec((1, rt, TOPK), lambda i: (i, 0, 0)),
            pl.BlockSpec((1, rt, TOPK), lambda i: (i, 0, 0)),
            pl.BlockSpec((4, EXPERTS), lambda i: (0, 0)),
        ],
        out_shape=[
            jax.ShapeDtypeStruct((tiles, rt, TOPK), jnp.int32),
            jax.ShapeDtypeStruct((tiles, rt, TOPK), jnp.int32),
            jax.ShapeDtypeStruct((tiles, rt, TOPK), jnp.float32),
            jax.ShapeDtypeStruct((4, EXPERTS), jnp.float32),
        ],
        scratch_shapes=[
            pltpu.VMEM((1, EXPERTS), jnp.float32),
            pltpu.VMEM((1, EXPERTS), jnp.float32),
            pltpu.VMEM((1, EXPERTS), jnp.float32),
            pltpu.VMEM((1, 1), jnp.float32),
        ],
    )(xf, w_router)


def _ffn_body(be_ref, xs_ref, w1_ref, w2_ref, wo_ref, g_ref, o_ref):
    xs = xs_ref[...]
    h1 = jnp.dot(xs, w1_ref[0], preferred_element_type=jnp.float32)
    hg = jnp.dot(xs, w2_ref[0], preferred_element_type=jnp.float32)
    act = h1 * jax.nn.sigmoid(h1) * hg
    out = jnp.dot(act, wo_ref[0], preferred_element_type=jnp.float32)
    o_ref[...] = out * g_ref[...]


def _grouped_ffn(xs, w_in, w_out, block_expert, row_gate_col, n_pad, fidx):
    r_tiles = n_pad // ROW_TILE
    grid_spec = pltpu.PrefetchScalarGridSpec(
        num_scalar_prefetch=1,
        grid=(r_tiles,),
        in_specs=[
            pl.BlockSpec((ROW_TILE, HIDDEN), lambda r, be: (r, 0)),
            pl.BlockSpec((1, HIDDEN, F_CHUNK), lambda r, be: (be[r], 0, fidx)),
            pl.BlockSpec((1, HIDDEN, F_CHUNK),
                         lambda r, be: (be[r], 0, fidx + F_SPLIT)),
            pl.BlockSpec((1, F_CHUNK, HIDDEN), lambda r, be: (be[r], fidx, 0)),
            pl.BlockSpec((ROW_TILE, 1), lambda r, be: (r, 0)),
        ],
        out_specs=pl.BlockSpec((ROW_TILE, HIDDEN), lambda r, be: (r, 0)),
    )
    return pl.pallas_call(
        _ffn_body,
        grid_spec=grid_spec,
        out_shape=jax.ShapeDtypeStruct((n_pad, HIDDEN), jnp.float32),
    )(block_expert, xs, w_in, w_in, w_out, row_gate_col)


def _combine_body(a_ref, b_ref, c_ref, d_ref, o_ref):
    o_ref[...] = (a_ref[...] + b_ref[...]) + (c_ref[...] + d_ref[...])


def _combine(picked0, picked1, t):
    tiles = t // ROW_TILE
    return pl.pallas_call(
        _combine_body,
        grid=(tiles,),
        in_specs=[
            pl.BlockSpec((ROW_TILE, HIDDEN), lambda i: (i, 0)),
            pl.BlockSpec((ROW_TILE, HIDDEN), lambda i: (i + tiles, 0)),
            pl.BlockSpec((ROW_TILE, HIDDEN), lambda i: (i, 0)),
            pl.BlockSpec((ROW_TILE, HIDDEN), lambda i: (i + tiles, 0)),
        ],
        out_specs=pl.BlockSpec((ROW_TILE, HIDDEN), lambda i: (i, 0)),
        out_shape=jax.ShapeDtypeStruct((t, HIDDEN), jnp.float32),
    )(picked0, picked0, picked1, picked1)


@jax.jit
def _moe(x, w_router, w_in, w_out):
    b, s, h = x.shape
    t = b * s
    xf = x.reshape(t, h)
    n_slots = t * TOPK
    n_pad = n_slots + EXPERTS * ROW_TILE

    ro_ranks, ro_te, ro_gate, stats = _router(xf, w_router, t)
    counts = stats[0].astype(jnp.int32)
    probs_sum, freq = stats[1], stats[2]
    switchloss = EXPERTS * jnp.sum(
        (probs_sum / probs_sum.sum()) * (freq / freq.sum()))
    loss = switchloss + 0.1 * (stats[3, 0] / t)

    te = ro_te.reshape(-1)
    ranks = ro_ranks.reshape(-1)
    aligned = ((counts + ROW_TILE - 1) // ROW_TILE) * ROW_TILE
    cum_aligned = jnp.cumsum(aligned)
    pad_start = cum_aligned - aligned
    pos = pad_start[te] + ranks
    slot_tok = jnp.arange(n_slots, dtype=jnp.int32) // TOPK
    row_token = jnp.zeros((n_pad,), jnp.int32).at[pos].set(slot_tok)
    row_gate = jnp.zeros((n_pad,), jnp.float32).at[pos].set(ro_gate.reshape(-1))
    r_tiles = n_pad // ROW_TILE
    block_expert = jnp.minimum(
        jnp.searchsorted(cum_aligned, jnp.arange(r_tiles, dtype=jnp.int32) * ROW_TILE,
                         side="right"),
        EXPERTS - 1).astype(jnp.int32)
    p0, p1 = pos[0::2], pos[1::2]
    comb_idx = jnp.concatenate([p0, p1, p0 + n_pad, p1 + n_pad])

    xs = _sc_gather_rows(xf, row_token)
    out0 = _grouped_ffn(xs, w_in, w_out, block_expert, row_gate[:, None],
                        n_pad, 0)
    out1 = _grouped_ffn(xs, w_in, w_out, block_expert, row_gate[:, None],
                        n_pad, 1)
    comb2 = jnp.concatenate([p0, p1])
    picked0 = _sc_gather_rows(out0, comb2)
    picked1 = _sc_gather_rows(out1, comb2)
    y = _combine(picked0, picked1, t)
    return y.reshape(b, s, h), loss


def kernel(x, W_router, W_in, W_out):
    return _moe(x, W_router, W_in, W_out)


# speedup vs baseline: 1.0001x; 1.0001x over previous
import jax
import jax.numpy as jnp
from jax import lax
from jax.experimental import pallas as pl
from jax.experimental.pallas import tpu as pltpu
from jax.experimental.pallas import tpu_sc as plsc

HIDDEN = 2048
FFN = 2048
EXPERTS = 8
TOPK = 2
ROW_TILE = 128

_NUM_WORKERS = 32
_GATHER_CHUNK = 16


def _sc_gather_rows(table, idx):
    n_rows = idx.shape[0]
    row_shape = table.shape[1:]
    npw = n_rows // _NUM_WORKERS
    assert npw * _NUM_WORKERS == n_rows and npw % 8 == 0
    nch = npw // _GATHER_CHUNK
    assert nch * _GATHER_CHUNK == npw and nch % 2 == 0

    mesh = plsc.VectorSubcoreMesh(core_axis_name="c", subcore_axis_name="s")

    def body(table_hbm, idx_hbm, out_hbm, idx_v, rows_a, rows_b, gsem, osem):
        wid = lax.axis_index("s") * 2 + lax.axis_index("c")
        base = wid * npw
        pltpu.sync_copy(idx_hbm.at[pl.ds(base, npw)], idx_v)

        @pl.loop(0, nch, step=2)
        def _pair(c):
            off0 = base + c * _GATHER_CHUNK
            off1 = off0 + _GATHER_CHUNK
            g0 = pltpu.async_copy(
                table_hbm.at[idx_v.at[pl.ds(c * _GATHER_CHUNK, _GATHER_CHUNK)]],
                rows_a, gsem)
            g1 = pltpu.async_copy(
                table_hbm.at[idx_v.at[pl.ds((c + 1) * _GATHER_CHUNK,
                                            _GATHER_CHUNK)]],
                rows_b, gsem)
            g0.wait()
            o0 = pltpu.async_copy(rows_a, out_hbm.at[pl.ds(off0, _GATHER_CHUNK)],
                                  osem)
            g1.wait()
            o1 = pltpu.async_copy(rows_b, out_hbm.at[pl.ds(off1, _GATHER_CHUNK)],
                                  osem)
            o0.wait()
            o1.wait()

    return pl.kernel(
        body,
        out_type=jax.ShapeDtypeStruct((n_rows,) + row_shape, table.dtype),
        mesh=mesh,
        scratch_types=[
            pltpu.VMEM((npw,), jnp.int32),
            pltpu.VMEM((_GATHER_CHUNK,) + row_shape, table.dtype),
            pltpu.VMEM((_GATHER_CHUNK,) + row_shape, table.dtype),
            pltpu.SemaphoreType.DMA,
            pltpu.SemaphoreType.DMA,
        ],
    )(table, idx)


F_SPLIT = 2
F_CHUNK = FFN // F_SPLIT


def _router_body(x_ref, wr_ref, ranks_ref, te_ref, gate_ref, stats_ref,
                 accc, accp, accf, accz):
    i = pl.program_id(0)

    @pl.when(i == 0)
    def _init():
        accc[...] = jnp.zeros_like(accc)
        accp[...] = jnp.zeros_like(accp)
        accf[...] = jnp.zeros_like(accf)
        accz[...] = jnp.zeros_like(accz)

    rows = x_ref.shape[0]
    logits = jnp.dot(x_ref[...], wr_ref[...],
                     preferred_element_type=jnp.float32)
    iota = lax.broadcasted_iota(jnp.int32, (rows, EXPERTS), 1)
    m1 = jnp.max(logits, axis=1, keepdims=True)
    e0 = jnp.min(jnp.where(logits == m1, iota, EXPERTS), axis=1, keepdims=True)
    oh0 = iota == e0
    masked = jnp.where(oh0, -jnp.inf, logits)
    m2 = jnp.max(masked, axis=1, keepdims=True)
    e1 = jnp.min(jnp.where(masked == m2, iota, EXPERTS), axis=1, keepdims=True)
    oh1 = iota == e1
    ed = jnp.exp(m2 - m1)
    g0 = 1.0 / (1.0 + ed)
    g1 = ed / (1.0 + ed)
    ex = jnp.exp(logits - m1)
    sex = jnp.sum(ex, axis=1, keepdims=True)
    lse = m1 + jnp.log(sex)
    r_iota = lax.broadcasted_iota(jnp.int32, (rows, rows), 0)
    c_iota = lax.broadcasted_iota(jnp.int32, (rows, rows), 1)
    tri = (c_iota < r_iota).astype(jnp.float32)
    oh0f = oh0.astype(jnp.float32)
    oh1f = oh1.astype(jnp.float32)
    prior = accc[...] + jnp.dot(tri, oh0f + oh1f,
                                preferred_element_type=jnp.float32)
    rank0 = jnp.sum(prior * oh0f, axis=1, keepdims=True)
    rank1 = jnp.sum(prior * oh1f, axis=1, keepdims=True)
    ranks_ref[0] = jnp.concatenate([rank0, rank1], axis=1).astype(jnp.int32)
    te_ref[0] = jnp.concatenate([e0, e1], axis=1)
    gate_ref[0] = jnp.concatenate([g0, g1], axis=1)
    accc[...] += jnp.sum(oh0f + oh1f, axis=0, keepdims=True)
    accp[...] += jnp.sum(ex / sex, axis=0, keepdims=True)
    accf[...] += jnp.sum(oh0f + jnp.where(g1 > 0, oh1f, 0.0), axis=0,
                         keepdims=True)
    accz[...] += jnp.sum(lse * lse).reshape(1, 1)

    @pl.when(i == pl.num_programs(0) - 1)
    def _fin():
        stats_ref[...] = jnp.concatenate(
            [accc[...], accp[...], accf[...],
             jnp.broadcast_to(accz[...], (1, EXPERTS))], axis=0)


def _router(xf, w_router, t):
    rt = 512
    tiles = t // rt
    return pl.pallas_call(
        _router_body,
        grid=(tiles,),
        in_specs=[
            pl.BlockSpec((rt, HIDDEN), lambda i: (i, 0)),
            pl.BlockSpec((HIDDEN, EXPERTS), lambda i: (0, 0)),
        ],
        out_specs=[
            pl.BlockSpec((1, rt, TOPK), lambda i: (i, 0, 0)),
            pl.BlockSpec((1, rt, TOPK), lambda i: (i, 0, 0)),
            pl.BlockSpec((1, rt, TOPK), lambda i: (i, 0, 0)),
            pl.BlockSpec((4, EXPERTS), lambda i: (0, 0)),
        ],
        out_shape=[
            jax.ShapeDtypeStruct((tiles, rt, TOPK), jnp.int32),
            jax.ShapeDtypeStruct((tiles, rt, TOPK), jnp.int32),
            jax.ShapeDtypeStruct((tiles, rt, TOPK), jnp.float32),
            jax.ShapeDtypeStruct((4, EXPERTS), jnp.float32),
        ],
        scratch_shapes=[
            pltpu.VMEM((1, EXPERTS), jnp.float32),
            pltpu.VMEM((1, EXPERTS), jnp.float32),
            pltpu.VMEM((1, EXPERTS), jnp.float32),
            pltpu.VMEM((1, 1), jnp.float32),
        ],
    )(xf, w_router)


def _ffn_body(be_ref, xs_ref, w1_ref, w2_ref, wo_ref, g_ref, o_ref):
    xs = xs_ref[...]
    h1 = jnp.dot(xs, w1_ref[0], preferred_element_type=jnp.float32)
    hg = jnp.dot(xs, w2_ref[0], preferred_element_type=jnp.float32)
    act = h1 * jax.nn.sigmoid(h1) * hg
    out = jnp.dot(act, wo_ref[0], preferred_element_type=jnp.float32)
    o_ref[...] = out * g_ref[...]


def _grouped_ffn(xs, w_in, w_out, block_expert, row_gate_col, n_pad, fidx):
    r_tiles = n_pad // ROW_TILE
    grid_spec = pltpu.PrefetchScalarGridSpec(
        num_scalar_prefetch=1,
        grid=(r_tiles,),
        in_specs=[
            pl.BlockSpec((ROW_TILE, HIDDEN), lambda r, be: (r, 0)),
            pl.BlockSpec((1, HIDDEN, F_CHUNK), lambda r, be: (be[r], 0, fidx)),
            pl.BlockSpec((1, HIDDEN, F_CHUNK),
                         lambda r, be: (be[r], 0, fidx + F_SPLIT)),
            pl.BlockSpec((1, F_CHUNK, HIDDEN), lambda r, be: (be[r], fidx, 0)),
            pl.BlockSpec((ROW_TILE, 1), lambda r, be: (r, 0)),
        ],
        out_specs=pl.BlockSpec((ROW_TILE, HIDDEN), lambda r, be: (r, 0)),
    )
    return pl.pallas_call(
        _ffn_body,
        grid_spec=grid_spec,
        out_shape=jax.ShapeDtypeStruct((n_pad, HIDDEN), jnp.float32),
    )(block_expert, xs, w_in, w_in, w_out, row_gate_col)


def _combine_body(a_ref, b_ref, c_ref, d_ref, o_ref):
    o_ref[...] = (a_ref[...] + b_ref[...]) + (c_ref[...] + d_ref[...])


def _combine(picked0, picked1, t):
    tiles = t // ROW_TILE
    return pl.pallas_call(
        _combine_body,
        grid=(tiles,),
        in_specs=[
            pl.BlockSpec((ROW_TILE, HIDDEN), lambda i: (i, 0)),
            pl.BlockSpec((ROW_TILE, HIDDEN), lambda i: (i + tiles, 0)),
            pl.BlockSpec((ROW_TILE, HIDDEN), lambda i: (i, 0)),
            pl.BlockSpec((ROW_TILE, HIDDEN), lambda i: (i + tiles, 0)),
        ],
        out_specs=pl.BlockSpec((ROW_TILE, HIDDEN), lambda i: (i, 0)),
        out_shape=jax.ShapeDtypeStruct((t, HIDDEN), jnp.float32),
    )(picked0, picked0, picked1, picked1)


@jax.jit
def _moe(x, w_router, w_in, w_out):
    b, s, h = x.shape
    t = b * s
    xf = x.reshape(t, h)
    n_slots = t * TOPK
    n_pad = n_slots + EXPERTS * ROW_TILE

    ro_ranks, ro_te, ro_gate, stats = _router(xf, w_router, t)
    counts = stats[0].astype(jnp.int32)
    probs_sum, freq = stats[1], stats[2]
    switchloss = EXPERTS * jnp.sum(
        (probs_sum / probs_sum.sum()) * (freq / freq.sum()))
    loss = switchloss + 0.1 * (stats[3, 0] / t)

    te = ro_te.reshape(-1)
    ranks = ro_ranks.reshape(-1)
    aligned = ((counts + ROW_TILE - 1) // ROW_TILE) * ROW_TILE
    cum_aligned = jnp.cumsum(aligned)
    pad_start = cum_aligned - aligned
    pos = pad_start[te] + ranks
    slot_tok = jnp.arange(n_slots, dtype=jnp.int32) // TOPK
    row_token = jnp.zeros((n_pad,), jnp.int32).at[pos].set(slot_tok)
    row_gate = jnp.zeros((n_pad,), jnp.float32).at[pos].set(ro_gate.reshape(-1))
    r_tiles = n_pad // ROW_TILE
    block_expert = jnp.minimum(
        jnp.searchsorted(cum_aligned, jnp.arange(r_tiles, dtype=jnp.int32) * ROW_TILE,
                         side="right"),
        EXPERTS - 1).astype(jnp.int32)
    p0, p1 = pos[0::2], pos[1::2]
    comb_idx = jnp.concatenate([p0, p1, p0 + n_pad, p1 + n_pad])

    xs = _sc_gather_rows(xf, row_token)
    out0 = _grouped_ffn(xs, w_in, w_out, block_expert, row_gate[:, None],
                        n_pad, 0)
    out1 = _grouped_ffn(xs, w_in, w_out, block_expert, row_gate[:, None],
                        n_pad, 1)
    comb2 = jnp.concatenate([p0, p1])
    picked0 = _sc_gather_rows(out0, comb2)
    picked1 = _sc_gather_rows(out1, comb2)
    y = _combine(picked0, picked1, t)
    return y.reshape(b, s, h), loss


def kernel(x, W_router, W_in, W_out):
    return _moe(x, W_router, W_in, W_out)
